# baseline (speedup 1.0000x reference)
import functools

import jax
import jax.numpy as jnp
from jax import lax
from jax.experimental import pallas as pl
from jax.experimental.pallas import tpu as pltpu
from jax.experimental.pallas import tpu_sc as plsc

B = 16384
D = 64
NC, NS = 2, 16
NW = NC * NS
BPW = B // NW
L = 16
CH = 128
NCH = BPW // CH
ROWS = 2048


def _sc_body(w01, w23, sex, tl, msi, st, age, out, nout,
             i0, i1, i2, i3, q0, q1, t0, t1, b00, b01, b10, b11,
             age_f, age_v, age_w,
             s0, s1, s2, s3, g00, g01, g10, g11, ws00, ws01, ws10, ws11,
             tsem, asem0, asem1):
    wid = lax.axis_index("s") * NC + lax.axis_index("c")
    base = wid * BPW
    idx_v = (i0, i1, i2, i3)
    q = (q0, q1)
    spm = (t0, t1)
    bufs = ((b00, b01), (b10, b11))
    gsems = ((g00, g01), (g10, g11))
    wsems = ((ws00, ws01), (ws10, ws11))

    @pl.when(lax.axis_index("s") == 0)
    def _():
        for w_ref, t_ref in zip((w01, w23), spm):
            pltpu.async_copy(w_ref, t_ref, tsem).wait()

    icopies = [
        pltpu.async_copy(src.at[pl.ds(base, BPW)], dst, sem)
        for src, dst, sem in zip((sex, tl, msi, st), idx_v, (s0, s1, s2, s3))
    ]
    asems = (asem0, asem1)
    acopy = pltpu.async_copy(age, age_f, asem0)
    for c in icopies:
        c.wait()

    for k in range(BPW // L):
        sl = pl.ds(L * k, L)
        q0[sl] = idx_v[0][sl] * 10 + idx_v[1][sl]
        q1[sl] = idx_v[2][sl] * 10 + idx_v[3][sl]

    plsc.subcore_barrier()

    def fire_gather(c, p):
        return pltpu.async_copy(
            spm[p].at[q[p].at[pl.ds(CH * c, CH)]],
            bufs[p][c % 2], gsems[p][c % 2])

    writes = {}
    gathers = {(0, p): fire_gather(0, p) for p in range(2)}

    acopy.wait()

    def stat_step(k, carry):
        s, s2 = carry
        v = age_f[pl.ds(L * k, L)]
        return s + v, s2 + v * v

    zeros = jnp.zeros((L,), jnp.float32)
    s, s2 = lax.fori_loop(0, B // L, stat_step, (zeros, zeros))
    mean = jnp.sum(s) * (1.0 / B)
    var = (jnp.sum(s2) - B * mean * mean) * (1.0 / (B - 1))
    var_v = jnp.full((L,), var, jnp.float32)
    y = plsc.bitcast(0x5F3759DF - (plsc.bitcast(var_v, jnp.int32) >> 1),
                     jnp.float32)
    for _ in range(4):
        y = y * (1.5 - 0.5 * var_v * y * y)
    mean_v = jnp.full((L,), mean, jnp.float32)

    lane = lax.iota(jnp.int32, L)
    zero = lane * 0
    avs = (age_v, age_w)
    was = []
    for j in range(NCH):
        av = avs[j % 2]
        if j >= 2:
            was[j - 2].wait()
        for k in range(CH // L):
            v = age_f[pl.ds(base + CH * j + L * k, L)]
            plsc.store_scatter(av, [lane + L * k, zero], (v - mean_v) * y)
        was.append(pltpu.async_copy(
            av, nout.at[pl.ds(base + CH * j, CH), :], asems[j % 2]))

    for c in range(NCH):
        for p in range(2):
            gathers[(c, p)].wait()
            writes[(c, p)] = pltpu.async_copy(
                bufs[p][c % 2],
                out.at[pl.ds(base + CH * c, CH), pl.ds(128 * p, 128)],
                wsems[p][c % 2])
        if c + 1 < NCH:
            for p in range(2):
                if c >= 1:
                    writes[(c - 1, p)].wait()
                gathers[(c + 1, p)] = fire_gather(c + 1, p)
    for p in range(2):
        writes[(NCH - 2, p)].wait()
        writes[(NCH - 1, p)].wait()
    was[NCH - 2].wait()
    was[NCH - 1].wait()
    pltpu.async_copy(nout.at[pl.ds(base, BPW), :],
                     out.at[pl.ds(base, BPW), pl.ds(4 * D, 1)],
                     asems[0]).wait()


_sc_lookup = functools.partial(
    pl.kernel,
    mesh=plsc.VectorSubcoreMesh(core_axis_name="c", subcore_axis_name="s"),
    out_type=(jax.ShapeDtypeStruct((B, 4 * D + 1), jnp.float32),
              jax.ShapeDtypeStruct((B, 1), jnp.float32)),
    scratch_types=[
        pltpu.VMEM((BPW,), jnp.int32),
        pltpu.VMEM((BPW,), jnp.int32),
        pltpu.VMEM((BPW,), jnp.int32),
        pltpu.VMEM((BPW,), jnp.int32),
        pltpu.VMEM((BPW,), jnp.int32),
        pltpu.VMEM((BPW,), jnp.int32),
        pltpu.VMEM_SHARED((100, 128), jnp.float32),
        pltpu.VMEM_SHARED((100, 128), jnp.float32),
        pltpu.VMEM((CH, 128), jnp.float32),
        pltpu.VMEM((CH, 128), jnp.float32),
        pltpu.VMEM((CH, 128), jnp.float32),
        pltpu.VMEM((CH, 128), jnp.float32),
        pltpu.VMEM((B,), jnp.float32),
        pltpu.VMEM((CH, 1), jnp.float32),
        pltpu.VMEM((CH, 1), jnp.float32),
        pltpu.SemaphoreType.DMA,
        pltpu.SemaphoreType.DMA,
        pltpu.SemaphoreType.DMA,
        pltpu.SemaphoreType.DMA,
        pltpu.SemaphoreType.DMA,
        pltpu.SemaphoreType.DMA,
        pltpu.SemaphoreType.DMA,
        pltpu.SemaphoreType.DMA,
        pltpu.SemaphoreType.DMA,
        pltpu.SemaphoreType.DMA,
        pltpu.SemaphoreType.DMA,
        pltpu.SemaphoreType.DMA,
        pltpu.SemaphoreType.DMA,
        pltpu.SemaphoreType.DMA,
        pltpu.SemaphoreType.DMA,
    ],
    compiler_params=pltpu.CompilerParams(use_tc_tiling_on_sc=True,
                                         needs_layout_passes=False),
)(_sc_body)


def _assemble_body(emb_ref, nage_ref, o_ref):
    o_ref[:, : 4 * D] = emb_ref[...]
    o_ref[:, 4 * D:] = nage_ref[...]


_assemble = pl.pallas_call(
    _assemble_body,
    grid=(B // ROWS,),
    in_specs=[
        pl.BlockSpec((ROWS, 4 * D), lambda i: (i, 0)),
        pl.BlockSpec((ROWS, 1), lambda i: (i, 0)),
    ],
    out_specs=pl.BlockSpec((ROWS, 4 * D + 1), lambda i: (i, 0)),
    out_shape=jax.ShapeDtypeStruct((B, 4 * D + 1), jnp.float32),
)


def kernel(sex, tumor_location, msi_status, stage, age,
           W_sex, W_tumor_location, W_msi_status, W_stage):
    w01 = jnp.concatenate(
        [jnp.repeat(W_sex, 10, axis=0), jnp.tile(W_tumor_location, (10, 1))],
        axis=1)
    w23 = jnp.concatenate(
        [jnp.repeat(W_msi_status, 10, axis=0), jnp.tile(W_stage, (10, 1))],
        axis=1)
    emb, nage = _sc_lookup(w01, w23, sex, tumor_location, msi_status, stage,
                           age)
    return emb


# speedup vs baseline: 4.0321x; 4.0321x over previous
import functools

import jax
import jax.numpy as jnp
from jax import lax
from jax.experimental import pallas as pl
from jax.experimental.pallas import tpu as pltpu
from jax.experimental.pallas import tpu_sc as plsc

B = 16384
D = 64
NC, NS = 2, 16
NW = NC * NS
BPW = B // NW
L = 16
CH = 128
NCH = BPW // CH
ROWS = 2048


def _sc_body(w01, w23, sex, tl, msi, st, age, out, nout,
             i0, i1, i2, i3, q0, q1, t0, t1, b00, b01, b10, b11,
             age_f, age_v, age_w,
             s0, s1, s2, s3, g00, g01, g10, g11, ws00, ws01, ws10, ws11,
             tsem, asem0, asem1):
    wid = lax.axis_index("s") * NC + lax.axis_index("c")
    base = wid * BPW
    idx_v = (i0, i1, i2, i3)
    q = (q0, q1)
    spm = (t0, t1)
    bufs = ((b00, b01), (b10, b11))
    gsems = ((g00, g01), (g10, g11))
    wsems = ((ws00, ws01), (ws10, ws11))

    @pl.when(lax.axis_index("s") == 0)
    def _():
        for w_ref, t_ref in zip((w01, w23), spm):
            pltpu.async_copy(w_ref, t_ref, tsem).wait()

    icopies = [
        pltpu.async_copy(src.at[pl.ds(base, BPW)], dst, sem)
        for src, dst, sem in zip((sex, tl, msi, st), idx_v, (s0, s1, s2, s3))
    ]
    asems = (asem0, asem1)
    acopy = pltpu.async_copy(age, age_f, asem0)
    for c in icopies:
        c.wait()

    for k in range(BPW // L):
        sl = pl.ds(L * k, L)
        q0[sl] = idx_v[0][sl] * 10 + idx_v[1][sl]
        q1[sl] = idx_v[2][sl] * 10 + idx_v[3][sl]

    plsc.subcore_barrier()

    def fire_gather(c, p):
        return pltpu.async_copy(
            spm[p].at[q[p].at[pl.ds(CH * c, CH)]],
            bufs[p][c % 2], gsems[p][c % 2])

    writes = {}
    gathers = {(0, p): fire_gather(0, p) for p in range(2)}

    acopy.wait()

    def stat_step(k, carry):
        s, s2 = carry
        v = age_f[pl.ds(L * k, L)]
        return s + v, s2 + v * v

    zeros = jnp.zeros((L,), jnp.float32)
    s, s2 = lax.fori_loop(0, B // L, stat_step, (zeros, zeros))
    mean = jnp.sum(s) * (1.0 / B)
    var = (jnp.sum(s2) - B * mean * mean) * (1.0 / (B - 1))
    var_v = jnp.full((L,), var, jnp.float32)
    y = plsc.bitcast(0x5F3759DF - (plsc.bitcast(var_v, jnp.int32) >> 1),
                     jnp.float32)
    for _ in range(4):
        y = y * (1.5 - 0.5 * var_v * y * y)
    mean_v = jnp.full((L,), mean, jnp.float32)

    lane = lax.iota(jnp.int32, L)
    zero = lane * 0
    avs = (age_v, age_w)
    was = []
    for j in range(NCH):
        av = avs[j % 2]
        if j >= 2:
            was[j - 2].wait()
        for k in range(CH // L):
            v = age_f[pl.ds(base + CH * j + L * k, L)]
            plsc.store_scatter(av, [lane + L * k, zero], (v - mean_v) * y)
        was.append(pltpu.async_copy(
            av, nout.at[pl.ds(base + CH * j, CH), :],
            asems[j % 2]))

    for c in range(NCH):
        for p in range(2):
            gathers[(c, p)].wait()
            writes[(c, p)] = pltpu.async_copy(
                bufs[p][c % 2],
                out.at[pl.ds(base + CH * c, CH), pl.ds(128 * p, 128)],
                wsems[p][c % 2])
        if c + 1 < NCH:
            for p in range(2):
                if c >= 1:
                    writes[(c - 1, p)].wait()
                gathers[(c + 1, p)] = fire_gather(c + 1, p)
    for p in range(2):
        writes[(NCH - 2, p)].wait()
        writes[(NCH - 1, p)].wait()
    was[NCH - 2].wait()
    was[NCH - 1].wait()


_sc_lookup = functools.partial(
    pl.kernel,
    mesh=plsc.VectorSubcoreMesh(core_axis_name="c", subcore_axis_name="s"),
    out_type=(jax.ShapeDtypeStruct((B, 4 * D + 1), jnp.float32),
              jax.ShapeDtypeStruct((B, 1), jnp.float32)),
    scratch_types=[
        pltpu.VMEM((BPW,), jnp.int32),
        pltpu.VMEM((BPW,), jnp.int32),
        pltpu.VMEM((BPW,), jnp.int32),
        pltpu.VMEM((BPW,), jnp.int32),
        pltpu.VMEM((BPW,), jnp.int32),
        pltpu.VMEM((BPW,), jnp.int32),
        pltpu.VMEM_SHARED((100, 128), jnp.float32),
        pltpu.VMEM_SHARED((100, 128), jnp.float32),
        pltpu.VMEM((CH, 128), jnp.float32),
        pltpu.VMEM((CH, 128), jnp.float32),
        pltpu.VMEM((CH, 128), jnp.float32),
        pltpu.VMEM((CH, 128), jnp.float32),
        pltpu.VMEM((B,), jnp.float32),
        pltpu.VMEM((CH, 1), jnp.float32),
        pltpu.VMEM((CH, 1), jnp.float32),
        pltpu.SemaphoreType.DMA,
        pltpu.SemaphoreType.DMA,
        pltpu.SemaphoreType.DMA,
        pltpu.SemaphoreType.DMA,
        pltpu.SemaphoreType.DMA,
        pltpu.SemaphoreType.DMA,
        pltpu.SemaphoreType.DMA,
        pltpu.SemaphoreType.DMA,
        pltpu.SemaphoreType.DMA,
        pltpu.SemaphoreType.DMA,
        pltpu.SemaphoreType.DMA,
        pltpu.SemaphoreType.DMA,
        pltpu.SemaphoreType.DMA,
        pltpu.SemaphoreType.DMA,
        pltpu.SemaphoreType.DMA,
    ],
    compiler_params=pltpu.CompilerParams(use_tc_tiling_on_sc=True,
                                         needs_layout_passes=False),
)(_sc_body)


def _assemble_body(emb_ref, nage_ref, o_ref):
    o_ref[:, : 4 * D] = emb_ref[:, : 4 * D]
    o_ref[:, 4 * D:] = nage_ref[...]


_assemble = pl.pallas_call(
    _assemble_body,
    grid=(B // ROWS,),
    in_specs=[
        pl.BlockSpec((ROWS, 4 * D + 1), lambda i: (i, 0)),
        pl.BlockSpec((ROWS, 1), lambda i: (i, 0)),
    ],
    out_specs=pl.BlockSpec((ROWS, 4 * D + 1), lambda i: (i, 0)),
    out_shape=jax.ShapeDtypeStruct((B, 4 * D + 1), jnp.float32),
)


def kernel(sex, tumor_location, msi_status, stage, age,
           W_sex, W_tumor_location, W_msi_status, W_stage):
    w01 = jnp.concatenate(
        [jnp.repeat(W_sex, 10, axis=0), jnp.tile(W_tumor_location, (10, 1))],
        axis=1)
    w23 = jnp.concatenate(
        [jnp.repeat(W_msi_status, 10, axis=0), jnp.tile(W_stage, (10, 1))],
        axis=1)
    emb, nage = _sc_lookup(w01, w23, sex, tumor_location, msi_status, stage,
                           age)
    return _assemble(emb, nage)


# speedup vs baseline: 4.1615x; 1.0321x over previous
import functools

import jax
import jax.numpy as jnp
from jax import lax
from jax.experimental import pallas as pl
from jax.experimental.pallas import tpu as pltpu
from jax.experimental.pallas import tpu_sc as plsc

B = 16384
D = 64
NC, NS = 2, 16
NW = NC * NS
BPW = B // NW
L = 16
CH = 128
NCH = BPW // CH
ROWS = 2048


def _sc_body(w01, w23, sex, tl, msi, st, age, out, nout,
             i0, i1, i2, i3, q0, q1, t0, t1, b00, b01, b10, b11,
             age_f, age_v, age_w,
             s0, s1, s2, s3, g00, g01, g10, g11, ws00, ws01, ws10, ws11,
             tsem, asem0, asem1):
    wid = lax.axis_index("s") * NC + lax.axis_index("c")
    base = wid * BPW
    idx_v = (i0, i1, i2, i3)
    q = (q0, q1)
    spm = (t0, t1)
    bufs = ((b00, b01), (b10, b11))
    gsems = ((g00, g01), (g10, g11))
    wsems = ((ws00, ws01), (ws10, ws11))

    @pl.when(lax.axis_index("s") == 0)
    def _():
        for w_ref, t_ref in zip((w01, w23), spm):
            pltpu.async_copy(w_ref, t_ref, tsem).wait()

    icopies = [
        pltpu.async_copy(src.at[pl.ds(base, BPW)], dst, sem)
        for src, dst, sem in zip((sex, tl, msi, st), idx_v, (s0, s1, s2, s3))
    ]
    asems = (asem0, asem1)
    acopy = pltpu.async_copy(age, age_f, asem0)
    for c in icopies:
        c.wait()

    for k in range(BPW // L):
        sl = pl.ds(L * k, L)
        q0[sl] = idx_v[0][sl] * 10 + idx_v[1][sl]
        q1[sl] = idx_v[2][sl] * 10 + idx_v[3][sl]

    plsc.subcore_barrier()

    def fire_gather(c, p):
        return pltpu.async_copy(
            spm[p].at[q[p].at[pl.ds(CH * c, CH)]],
            bufs[p][c % 2], gsems[p][c % 2])

    writes = {}
    gathers = {(0, p): fire_gather(0, p) for p in range(2)}

    acopy.wait()

    def stat_step(k, carry):
        s, s2 = carry
        v = age_f[pl.ds(L * k, L)]
        return s + v, s2 + v * v

    zeros = jnp.zeros((L,), jnp.float32)
    s, s2 = lax.fori_loop(0, B // L, stat_step, (zeros, zeros))
    mean = jnp.sum(s) * (1.0 / B)
    var = (jnp.sum(s2) - B * mean * mean) * (1.0 / (B - 1))
    var_v = jnp.full((L,), var, jnp.float32)
    y = plsc.bitcast(0x5F3759DF - (plsc.bitcast(var_v, jnp.int32) >> 1),
                     jnp.float32)
    for _ in range(4):
        y = y * (1.5 - 0.5 * var_v * y * y)
    mean_v = jnp.full((L,), mean, jnp.float32)

    lane = lax.iota(jnp.int32, L)
    zero = lane * 0
    avs = (age_v, age_w)
    was = []
    for j in range(NCH):
        av = avs[j % 2]
        if j >= 2:
            was[j - 2].wait()
        for k in range(CH // L):
            v = age_f[pl.ds(base + CH * j + L * k, L)]
            plsc.store_scatter(av, [lane + L * k, zero], (v - mean_v) * y)
        was.append(pltpu.async_copy(
            av, nout.at[pl.ds(base + CH * j, CH), :],
            asems[j % 2]))

    for c in range(NCH):
        for p in range(2):
            gathers[(c, p)].wait()
            writes[(c, p)] = pltpu.async_copy(
                bufs[p][c % 2],
                out.at[pl.ds(base + CH * c, CH), pl.ds(128 * p, 128)],
                wsems[p][c % 2])
        if c + 1 < NCH:
            for p in range(2):
                if c >= 1:
                    writes[(c - 1, p)].wait()
                gathers[(c + 1, p)] = fire_gather(c + 1, p)
    for p in range(2):
        writes[(NCH - 2, p)].wait()
        writes[(NCH - 1, p)].wait()
    was[NCH - 2].wait()
    was[NCH - 1].wait()


_sc_lookup = functools.partial(
    pl.kernel,
    mesh=plsc.VectorSubcoreMesh(core_axis_name="c", subcore_axis_name="s"),
    out_type=(jax.ShapeDtypeStruct((B, 4 * D), jnp.float32),
              jax.ShapeDtypeStruct((B, 1), jnp.float32)),
    scratch_types=[
        pltpu.VMEM((BPW,), jnp.int32),
        pltpu.VMEM((BPW,), jnp.int32),
        pltpu.VMEM((BPW,), jnp.int32),
        pltpu.VMEM((BPW,), jnp.int32),
        pltpu.VMEM((BPW,), jnp.int32),
        pltpu.VMEM((BPW,), jnp.int32),
        pltpu.VMEM_SHARED((100, 128), jnp.float32),
        pltpu.VMEM_SHARED((100, 128), jnp.float32),
        pltpu.VMEM((CH, 128), jnp.float32),
        pltpu.VMEM((CH, 128), jnp.float32),
        pltpu.VMEM((CH, 128), jnp.float32),
        pltpu.VMEM((CH, 128), jnp.float32),
        pltpu.VMEM((B,), jnp.float32),
        pltpu.VMEM((CH, 1), jnp.float32),
        pltpu.VMEM((CH, 1), jnp.float32),
        pltpu.SemaphoreType.DMA,
        pltpu.SemaphoreType.DMA,
        pltpu.SemaphoreType.DMA,
        pltpu.SemaphoreType.DMA,
        pltpu.SemaphoreType.DMA,
        pltpu.SemaphoreType.DMA,
        pltpu.SemaphoreType.DMA,
        pltpu.SemaphoreType.DMA,
        pltpu.SemaphoreType.DMA,
        pltpu.SemaphoreType.DMA,
        pltpu.SemaphoreType.DMA,
        pltpu.SemaphoreType.DMA,
        pltpu.SemaphoreType.DMA,
        pltpu.SemaphoreType.DMA,
        pltpu.SemaphoreType.DMA,
    ],
    compiler_params=pltpu.CompilerParams(use_tc_tiling_on_sc=True,
                                         needs_layout_passes=False),
)(_sc_body)


def _assemble_body(emb_ref, nage_ref, o_ref):
    o_ref[:, : 4 * D] = emb_ref[...]
    o_ref[:, 4 * D:] = nage_ref[...]


_assemble = pl.pallas_call(
    _assemble_body,
    grid=(B // ROWS,),
    in_specs=[
        pl.BlockSpec((ROWS, 4 * D), lambda i: (i, 0)),
        pl.BlockSpec((ROWS, 1), lambda i: (i, 0)),
    ],
    out_specs=pl.BlockSpec((ROWS, 4 * D + 1), lambda i: (i, 0)),
    out_shape=jax.ShapeDtypeStruct((B, 4 * D + 1), jnp.float32),
)


def kernel(sex, tumor_location, msi_status, stage, age,
           W_sex, W_tumor_location, W_msi_status, W_stage):
    w01 = jnp.concatenate(
        [jnp.repeat(W_sex, 10, axis=0), jnp.tile(W_tumor_location, (10, 1))],
        axis=1)
    w23 = jnp.concatenate(
        [jnp.repeat(W_msi_status, 10, axis=0), jnp.tile(W_stage, (10, 1))],
        axis=1)
    emb, nage = _sc_lookup(w01, w23, sex, tumor_location, msi_status, stage,
                           age)
    return _assemble(emb, nage)


# speedup vs baseline: 4.2100x; 1.0116x over previous
import functools

import jax
import jax.numpy as jnp
from jax import lax
from jax.experimental import pallas as pl
from jax.experimental.pallas import tpu as pltpu
from jax.experimental.pallas import tpu_sc as plsc

B = 16384
D = 64
NC, NS = 2, 16
NW = NC * NS
BPW = B // NW
L = 16
CH = 128
NCH = BPW // CH
ROWS = 4096


def _sc_body(w01, w23, sex, tl, msi, st, age, out, nout,
             i0, i1, i2, i3, q0, q1, t0, t1, b00, b01, b10, b11,
             age_f, age_v, age_w,
             s0, s1, s2, s3, g00, g01, g10, g11, ws00, ws01, ws10, ws11,
             tsem, asem0, asem1):
    wid = lax.axis_index("s") * NC + lax.axis_index("c")
    base = wid * BPW
    idx_v = (i0, i1, i2, i3)
    q = (q0, q1)
    spm = (t0, t1)
    bufs = ((b00, b01), (b10, b11))
    gsems = ((g00, g01), (g10, g11))
    wsems = ((ws00, ws01), (ws10, ws11))

    @pl.when(lax.axis_index("s") == 0)
    def _():
        for w_ref, t_ref in zip((w01, w23), spm):
            pltpu.async_copy(w_ref, t_ref, tsem).wait()

    icopies = [
        pltpu.async_copy(src.at[pl.ds(base, BPW)], dst, sem)
        for src, dst, sem in zip((sex, tl, msi, st), idx_v, (s0, s1, s2, s3))
    ]
    asems = (asem0, asem1)
    acopy = pltpu.async_copy(age, age_f, asem0)
    for c in icopies:
        c.wait()

    for k in range(BPW // L):
        sl = pl.ds(L * k, L)
        q0[sl] = idx_v[0][sl] * 10 + idx_v[1][sl]
        q1[sl] = idx_v[2][sl] * 10 + idx_v[3][sl]

    plsc.subcore_barrier()

    def fire_gather(c, p):
        return pltpu.async_copy(
            spm[p].at[q[p].at[pl.ds(CH * c, CH)]],
            bufs[p][c % 2], gsems[p][c % 2])

    writes = {}
    gathers = {(0, p): fire_gather(0, p) for p in range(2)}

    acopy.wait()

    def stat_step(k, carry):
        s, s2 = carry
        v = age_f[pl.ds(L * k, L)]
        return s + v, s2 + v * v

    zeros = jnp.zeros((L,), jnp.float32)
    s, s2 = lax.fori_loop(0, B // L, stat_step, (zeros, zeros))
    mean = jnp.sum(s) * (1.0 / B)
    var = (jnp.sum(s2) - B * mean * mean) * (1.0 / (B - 1))
    var_v = jnp.full((L,), var, jnp.float32)
    y = plsc.bitcast(0x5F3759DF - (plsc.bitcast(var_v, jnp.int32) >> 1),
                     jnp.float32)
    for _ in range(4):
        y = y * (1.5 - 0.5 * var_v * y * y)
    mean_v = jnp.full((L,), mean, jnp.float32)

    lane = lax.iota(jnp.int32, L)
    zero = lane * 0
    avs = (age_v, age_w)
    was = []
    for j in range(NCH):
        av = avs[j % 2]
        if j >= 2:
            was[j - 2].wait()
        for k in range(CH // L):
            v = age_f[pl.ds(base + CH * j + L * k, L)]
            plsc.store_scatter(av, [lane + L * k, zero], (v - mean_v) * y)
        was.append(pltpu.async_copy(
            av, nout.at[pl.ds(base + CH * j, CH), :],
            asems[j % 2]))

    for c in range(NCH):
        for p in range(2):
            gathers[(c, p)].wait()
            writes[(c, p)] = pltpu.async_copy(
                bufs[p][c % 2],
                out.at[pl.ds(base + CH * c, CH), pl.ds(128 * p, 128)],
                wsems[p][c % 2])
        if c + 1 < NCH:
            for p in range(2):
                if c >= 1:
                    writes[(c - 1, p)].wait()
                gathers[(c + 1, p)] = fire_gather(c + 1, p)
    for p in range(2):
        writes[(NCH - 2, p)].wait()
        writes[(NCH - 1, p)].wait()
    was[NCH - 2].wait()
    was[NCH - 1].wait()


_sc_lookup = functools.partial(
    pl.kernel,
    mesh=plsc.VectorSubcoreMesh(core_axis_name="c", subcore_axis_name="s"),
    out_type=(jax.ShapeDtypeStruct((B, 4 * D), jnp.float32),
              jax.ShapeDtypeStruct((B, 1), jnp.float32)),
    scratch_types=[
        pltpu.VMEM((BPW,), jnp.int32),
        pltpu.VMEM((BPW,), jnp.int32),
        pltpu.VMEM((BPW,), jnp.int32),
        pltpu.VMEM((BPW,), jnp.int32),
        pltpu.VMEM((BPW,), jnp.int32),
        pltpu.VMEM((BPW,), jnp.int32),
        pltpu.VMEM_SHARED((100, 128), jnp.float32),
        pltpu.VMEM_SHARED((100, 128), jnp.float32),
        pltpu.VMEM((CH, 128), jnp.float32),
        pltpu.VMEM((CH, 128), jnp.float32),
        pltpu.VMEM((CH, 128), jnp.float32),
        pltpu.VMEM((CH, 128), jnp.float32),
        pltpu.VMEM((B,), jnp.float32),
        pltpu.VMEM((CH, 1), jnp.float32),
        pltpu.VMEM((CH, 1), jnp.float32),
        pltpu.SemaphoreType.DMA,
        pltpu.SemaphoreType.DMA,
        pltpu.SemaphoreType.DMA,
        pltpu.SemaphoreType.DMA,
        pltpu.SemaphoreType.DMA,
        pltpu.SemaphoreType.DMA,
        pltpu.SemaphoreType.DMA,
        pltpu.SemaphoreType.DMA,
        pltpu.SemaphoreType.DMA,
        pltpu.SemaphoreType.DMA,
        pltpu.SemaphoreType.DMA,
        pltpu.SemaphoreType.DMA,
        pltpu.SemaphoreType.DMA,
        pltpu.SemaphoreType.DMA,
        pltpu.SemaphoreType.DMA,
    ],
    compiler_params=pltpu.CompilerParams(use_tc_tiling_on_sc=True,
                                         needs_layout_passes=False),
)(_sc_body)


def _assemble_body(emb_ref, nage_ref, o_ref):
    o_ref[:, : 4 * D] = emb_ref[...]
    o_ref[:, 4 * D:] = nage_ref[...]


_assemble = pl.pallas_call(
    _assemble_body,
    grid=(B // ROWS,),
    in_specs=[
        pl.BlockSpec((ROWS, 4 * D), lambda i: (i, 0)),
        pl.BlockSpec((ROWS, 1), lambda i: (i, 0)),
    ],
    out_specs=pl.BlockSpec((ROWS, 4 * D + 1), lambda i: (i, 0)),
    out_shape=jax.ShapeDtypeStruct((B, 4 * D + 1), jnp.float32),
)


def kernel(sex, tumor_location, msi_status, stage, age,
           W_sex, W_tumor_location, W_msi_status, W_stage):
    w01 = jnp.concatenate(
        [jnp.repeat(W_sex, 10, axis=0), jnp.tile(W_tumor_location, (10, 1))],
        axis=1)
    w23 = jnp.concatenate(
        [jnp.repeat(W_msi_status, 10, axis=0), jnp.tile(W_stage, (10, 1))],
        axis=1)
    emb, nage = _sc_lookup(w01, w23, sex, tumor_location, msi_status, stage,
                           age)
    return _assemble(emb, nage)
